# baseline (speedup 1.0000x reference)
import functools

import jax
import jax.numpy as jnp
from jax import lax
from jax.experimental import pallas as pl
from jax.experimental.pallas import tpu as pltpu
from jax.experimental.pallas import tpu_sc as plsc

N = 10000
NPAD = 10240
E = 320000
EPAD = 327680
D = 128
G = 16
NC = 2
NS = 16
NW = NC * NS
CH = (EPAD // 128) // NW
CH0 = 32
CH1 = 160 - CH0
RPT = NPAD // NS
BR = 1024
NBLK = NPAD // BR

_f32 = jnp.float32
_i32 = jnp.int32

_mesh = plsc.VectorSubcoreMesh(core_axis_name="c", subcore_axis_name="s")


def _prep_body(src_h, dst_h, src2_h, dst2_h, degp_h, degsh, *sc):
    sbs = sc[0:8]
    sbd = sc[8:16]
    ones = sc[16]
    zb = sc[17]
    isem = sc[18:26]
    wsem = sc[26:34]
    c = lax.axis_index("c")
    s = lax.axis_index("s")
    wid = c * NS + s

    def _off(ch):
        return (wid * CH + ch) * 128

    def _fire_idx(ch):
        b = ch % 8
        return (pltpu.async_copy(src_h.at[pl.ds(_off(ch), 128)], sbs[b], isem[b]),
                pltpu.async_copy(dst_h.at[pl.ds(_off(ch), 128)], sbd[b], isem[b]))

    def _fill(i, _):
        for j in range(8):
            ones[i, pl.ds(j * 16, 16)] = jnp.ones((16,), _f32)
            zb[i, pl.ds(j * 16, 16)] = jnp.zeros((16,), _f32)
        return 0
    lax.fori_loop(0, 128, _fill, 0)

    row0 = s * RPT
    for t in range(RPT // 128):
        pltpu.sync_copy(zb, degsh.at[pl.ds(row0 + t * 128, 128)])
    plsc.subcore_barrier()

    hidx = {}
    hw = {}
    for ch in range(4):
        hidx[ch] = _fire_idx(ch)
    for ch in range(CH):
        b = ch % 8
        if ch >= 4:
            for h in hw.pop(ch - 4):
                h.wait()
        if ch + 4 < CH:
            hidx[ch + 4] = _fire_idx(ch + 4)
        for h in hidx.pop(ch):
            h.wait()
        for j in range(8):
            sl = pl.ds(j * 16, 16)
            sv = sbs[b][sl]
            dv = sbd[b][sl]
            m = sv == dv
            sbs[b][sl] = jnp.where(m, jnp.full((16,), N, _i32), sv)
            sbd[b][sl] = jnp.where(m, jnp.full((16,), N + 1, _i32), dv)
        hw[ch] = (
            pltpu.async_copy(sbs[b], src2_h.at[pl.ds(_off(ch), 128)], wsem[b]),
            pltpu.async_copy(sbd[b], dst2_h.at[pl.ds(_off(ch), 128)], wsem[b]),
        )
        pltpu.sync_copy(ones, degsh.at[sbs[b]], add=True)
    for ch in sorted(hw):
        for h in hw[ch]:
            h.wait()

    plsc.subcore_barrier()
    pltpu.sync_copy(degsh.at[pl.ds(row0, RPT)],
                    degp_h.at[pl.ds(c * NPAD + row0, RPT)])


_prep = functools.partial(
    pl.kernel,
    out_type=(jax.ShapeDtypeStruct((EPAD,), _i32),
              jax.ShapeDtypeStruct((EPAD,), _i32),
              jax.ShapeDtypeStruct((2 * NPAD, 128), _f32)),
    mesh=_mesh,
    scratch_types=(
        [pltpu.VMEM_SHARED((NPAD, 128), _f32)]
        + [pltpu.VMEM((128,), _i32) for _ in range(16)]
        + [pltpu.VMEM((128, 128), _f32) for _ in range(2)]
        + [pltpu.SemaphoreType.DMA for _ in range(16)]
    ),
)(_prep_body)


def _spmm_body(src_h, dst_h, y_h, z_h, zsh, *sc):
    six = sc[0:8]
    dix = sc[8:16]
    gb = sc[16:18]
    isem = sc[18:26]
    gsem = sc[26:28]
    ssem = sc[28:30]
    c = lax.axis_index("c")
    s = lax.axis_index("s")

    ch_c = jnp.where(c == 0, CH0, CH1)
    base = jnp.where(c == 0, s * CH0, NS * CH0 + s * CH1)

    def _zrow(i, _):
        for j in range(8):
            gb[0][i, pl.ds(j * 16, 16)] = jnp.zeros((16,), _f32)
        return 0
    lax.fori_loop(0, 128, _zrow, 0)
    row0 = s * RPT
    for t in range(RPT // 128):
        pltpu.sync_copy(gb[0], zsh.at[pl.ds(row0 + t * 128, 128)])
    plsc.subcore_barrier()

    def _super(t, _):
        off0 = (base + t * 8) * 128
        hidx = [
            (pltpu.async_copy(src_h.at[pl.ds(off0 + b * 128, 128)], six[b], isem[b]),
             pltpu.async_copy(dst_h.at[pl.ds(off0 + b * 128, 128)], dix[b], isem[b]))
            for b in range(8)
        ]
        hg = {}
        hs = {}
        for b in range(8):
            g = b % 2
            if b >= 2:
                hs.pop(b - 2).wait()
            for h in hidx[b]:
                h.wait()
            hg[b] = pltpu.async_copy(y_h.at[six[b]], gb[g], gsem[g])
            if b >= 1:
                hg.pop(b - 1).wait()
                hs[b - 1] = pltpu.async_copy(
                    gb[(b - 1) % 2], zsh.at[dix[b - 1]], ssem[(b - 1) % 2],
                    add=True)
        hg.pop(7).wait()
        hs[7] = pltpu.async_copy(gb[1], zsh.at[dix[7]], ssem[1], add=True)
        hs.pop(6).wait()
        hs.pop(7).wait()
        return 0
    lax.fori_loop(0, ch_c // 8, _super, 0)

    plsc.subcore_barrier()
    pltpu.sync_copy(zsh.at[pl.ds(row0, RPT)],
                    z_h.at[pl.ds(c * NPAD + row0, RPT)])


_spmm = functools.partial(
    pl.kernel,
    out_type=jax.ShapeDtypeStruct((2 * NPAD, D), _f32),
    mesh=_mesh,
    scratch_types=(
        [pltpu.VMEM_SHARED((NPAD, D), _f32)]
        + [pltpu.VMEM((128,), _i32) for _ in range(16)]
        + [pltpu.VMEM((128, D), _f32) for _ in range(2)]
        + [pltpu.SemaphoreType.DMA for _ in range(12)]
    ),
)(_spmm_body)


def _t1_body(dp0, dp1, x_ref, y0_ref, dis_ref, inv_ref):
    i = pl.program_id(0)
    deg = dp0[:, 0:1] + dp1[:, 0:1] + 1.0
    rows = i * BR + lax.broadcasted_iota(_i32, (BR, 1), 0)
    mask = rows < N
    dis = jnp.where(mask, lax.rsqrt(deg), 0.0)
    inv = jnp.where(mask, 1.0 / deg, 0.0)
    y0_ref[...] = x_ref[...] * dis
    dis_ref[...] = dis
    inv_ref[...] = inv


def _t1(degp, x_p):
    return pl.pallas_call(
        _t1_body,
        grid=(NBLK,),
        in_specs=[
            pl.BlockSpec((BR, 128), lambda i: (i, 0)),
            pl.BlockSpec((BR, 128), lambda i: (i + NBLK, 0)),
            pl.BlockSpec((BR, D), lambda i: (i, 0)),
        ],
        out_specs=[
            pl.BlockSpec((BR, D), lambda i: (i, 0)),
            pl.BlockSpec((BR, 1), lambda i: (i, 0)),
            pl.BlockSpec((BR, 1), lambda i: (i, 0)),
        ],
        out_shape=[
            jax.ShapeDtypeStruct((NPAD, D), _f32),
            jax.ShapeDtypeStruct((NPAD, 1), _f32),
            jax.ShapeDtypeStruct((NPAD, 1), _f32),
        ],
    )(degp, degp, x_p)


def _t2_body(z0, z1, y, inv, out):
    out[...] = (z0[...] + z1[...] + y[...]) * inv[...]


def _t2(z, y, inv):
    return pl.pallas_call(
        _t2_body,
        grid=(NBLK,),
        in_specs=[
            pl.BlockSpec((BR, D), lambda i: (i, 0)),
            pl.BlockSpec((BR, D), lambda i: (i + NBLK, 0)),
            pl.BlockSpec((BR, D), lambda i: (i, 0)),
            pl.BlockSpec((BR, 1), lambda i: (i, 0)),
        ],
        out_specs=pl.BlockSpec((BR, D), lambda i: (i, 0)),
        out_shape=jax.ShapeDtypeStruct((NPAD, D), _f32),
    )(z, z, y, inv)


def _t3_body(z0, z1, y, dis, w, b, out):
    x2 = dis[...] * (z0[...] + z1[...] + y[...])
    h = lax.dot_general(x2, w[...], (((1,), (1,)), ((), ())),
                        preferred_element_type=_f32)
    h = jnp.maximum(h + b[...], 0.0)
    out[...] = dis[...] * h


def _t3(z, y, dis, w, b):
    return pl.pallas_call(
        _t3_body,
        grid=(NBLK,),
        in_specs=[
            pl.BlockSpec((BR, D), lambda i: (i, 0)),
            pl.BlockSpec((BR, D), lambda i: (i + NBLK, 0)),
            pl.BlockSpec((BR, D), lambda i: (i, 0)),
            pl.BlockSpec((BR, 1), lambda i: (i, 0)),
            pl.BlockSpec((D, D), lambda i: (0, 0)),
            pl.BlockSpec((1, D), lambda i: (0, 0)),
        ],
        out_specs=pl.BlockSpec((BR, D), lambda i: (i, 0)),
        out_shape=jax.ShapeDtypeStruct((NPAD, D), _f32),
    )(z, z, y, dis, w, b)


def _t5_body(z0, z1, y, dis, w, b, bids, wfc, bfc, out, acc):
    i = pl.program_id(0)

    @pl.when(i == 0)
    def _():
        acc[...] = jnp.zeros_like(acc)

    x2 = dis[...] * (z0[...] + z1[...] + y[...])
    h = lax.dot_general(x2, w[...], (((1,), (1,)), ((), ())),
                        preferred_element_type=_f32)
    h = jnp.maximum(h + b[...], 0.0)
    ids = bids[0, 0, :]
    oh = (ids[None, :] == lax.broadcasted_iota(_i32, (G, BR), 0)).astype(_f32)
    acc[...] += lax.dot_general(oh, h, (((1,), (0,)), ((), ())),
                                preferred_element_type=_f32)

    @pl.when(i == pl.num_programs(0) - 1)
    def _():
        out[...] = lax.dot_general(acc[...], wfc[...],
                                   (((1,), (1,)), ((), ())),
                                   preferred_element_type=_f32) + bfc[...]


def _t5(z, y, dis, w, b, bids, wfc, bfc):
    return pl.pallas_call(
        _t5_body,
        grid=(NBLK,),
        in_specs=[
            pl.BlockSpec((BR, D), lambda i: (i, 0)),
            pl.BlockSpec((BR, D), lambda i: (i + NBLK, 0)),
            pl.BlockSpec((BR, D), lambda i: (i, 0)),
            pl.BlockSpec((BR, 1), lambda i: (i, 0)),
            pl.BlockSpec((D, D), lambda i: (0, 0)),
            pl.BlockSpec((1, D), lambda i: (0, 0)),
            pl.BlockSpec((1, 1, BR), lambda i: (i, 0, 0)),
            pl.BlockSpec((3, D), lambda i: (0, 0)),
            pl.BlockSpec((1, 3), lambda i: (0, 0)),
        ],
        out_specs=pl.BlockSpec((G, 3), lambda i: (0, 0)),
        out_shape=jax.ShapeDtypeStruct((G, 3), _f32),
        scratch_shapes=[pltpu.VMEM((G, D), _f32)],
    )(z, z, y, dis, w, b, bids, wfc, bfc)


def kernel(x, edge_index, batch, W1, b1, W2, b2, Wfc, bfc):
    src = edge_index[0]
    dst = edge_index[1]
    src_p = jnp.concatenate([src, jnp.full((EPAD - E,), N, _i32)])
    dst_p = jnp.concatenate([dst, jnp.full((EPAD - E,), N + 1, _i32)])
    x_p = jnp.pad(x, ((0, NPAD - N), (0, 0)))
    batch_p = jnp.concatenate(
        [batch, jnp.full((NPAD - N,), G, _i32)]).reshape(NBLK, 1, BR)
    b1r = b1.reshape(1, D)
    b2r = b2.reshape(1, D)
    bfcr = bfc.reshape(1, 3)

    src2, dst2, degp = _prep(src_p, dst_p)
    y0, dis, inv = _t1(degp, x_p)

    z = _spmm(src2, dst2, y0)
    y1 = _t2(z, y0, inv)
    z = _spmm(src2, dst2, y1)
    y = _t3(z, y1, dis, W1, b1r)

    z = _spmm(src2, dst2, y)
    y1 = _t2(z, y, inv)
    z = _spmm(src2, dst2, y1)
    return _t5(z, y1, dis, W2, b2r, batch_p, Wfc, bfcr)


# speedup vs baseline: 1.1932x; 1.1932x over previous
import functools

import jax
import jax.numpy as jnp
from jax import lax
from jax.experimental import pallas as pl
from jax.experimental.pallas import tpu as pltpu
from jax.experimental.pallas import tpu_sc as plsc

N = 10000
NPAD = 10240
E = 320000
EPAD = 327680
D = 128
G = 16
NC = 2
NS = 16
NW = NC * NS
CH = (EPAD // 128) // NW
CH0 = 128
CH1 = 160 - CH0
RPT = NPAD // NS
BR = 1024
NBLK = NPAD // BR

_f32 = jnp.float32
_i32 = jnp.int32

_mesh = plsc.VectorSubcoreMesh(core_axis_name="c", subcore_axis_name="s")


def _prep_body(src_h, dst_h, src2_h, dst2_h, degp_h, degsh, *sc):
    sbs = sc[0:8]
    sbd = sc[8:16]
    ones = sc[16]
    zb = sc[17]
    isem = sc[18:26]
    wsem = sc[26:34]
    c = lax.axis_index("c")
    s = lax.axis_index("s")
    wid = c * NS + s

    def _off(ch):
        return (wid * CH + ch) * 128

    def _fire_idx(ch):
        b = ch % 8
        return (pltpu.async_copy(src_h.at[pl.ds(_off(ch), 128)], sbs[b], isem[b]),
                pltpu.async_copy(dst_h.at[pl.ds(_off(ch), 128)], sbd[b], isem[b]))

    def _fill(i, _):
        for j in range(8):
            ones[i, pl.ds(j * 16, 16)] = jnp.ones((16,), _f32)
            zb[i, pl.ds(j * 16, 16)] = jnp.zeros((16,), _f32)
        return 0
    lax.fori_loop(0, 128, _fill, 0)

    row0 = s * RPT
    for t in range(RPT // 128):
        pltpu.sync_copy(zb, degsh.at[pl.ds(row0 + t * 128, 128)])
    plsc.subcore_barrier()

    hidx = {}
    hw = {}
    for ch in range(4):
        hidx[ch] = _fire_idx(ch)
    for ch in range(CH):
        b = ch % 8
        if ch >= 4:
            for h in hw.pop(ch - 4):
                h.wait()
        if ch + 4 < CH:
            hidx[ch + 4] = _fire_idx(ch + 4)
        for h in hidx.pop(ch):
            h.wait()
        for j in range(8):
            sl = pl.ds(j * 16, 16)
            sv = sbs[b][sl]
            dv = sbd[b][sl]
            m = sv == dv
            sbs[b][sl] = jnp.where(m, jnp.full((16,), N, _i32), sv)
            sbd[b][sl] = jnp.where(m, jnp.full((16,), N + 1, _i32), dv)
        hw[ch] = (
            pltpu.async_copy(sbs[b], src2_h.at[pl.ds(_off(ch), 128)], wsem[b]),
            pltpu.async_copy(sbd[b], dst2_h.at[pl.ds(_off(ch), 128)], wsem[b]),
        )
        pltpu.sync_copy(ones, degsh.at[sbs[b]], add=True)
    for ch in sorted(hw):
        for h in hw[ch]:
            h.wait()

    plsc.subcore_barrier()
    pltpu.sync_copy(degsh.at[pl.ds(row0, RPT)],
                    degp_h.at[pl.ds(c * NPAD + row0, RPT)])


_prep = functools.partial(
    pl.kernel,
    out_type=(jax.ShapeDtypeStruct((EPAD,), _i32),
              jax.ShapeDtypeStruct((EPAD,), _i32),
              jax.ShapeDtypeStruct((2 * NPAD, 128), _f32)),
    mesh=_mesh,
    scratch_types=(
        [pltpu.VMEM_SHARED((NPAD, 128), _f32)]
        + [pltpu.VMEM((128,), _i32) for _ in range(16)]
        + [pltpu.VMEM((128, 128), _f32) for _ in range(2)]
        + [pltpu.SemaphoreType.DMA for _ in range(16)]
    ),
)(_prep_body)


def _spmm_body(src_h, dst_h, y_h, z_h, zsh, *sc):
    six = sc[0:8]
    dix = sc[8:16]
    gb = sc[16:18]
    isem = sc[18:26]
    gsem = sc[26:28]
    ssem = sc[28:30]
    c = lax.axis_index("c")
    s = lax.axis_index("s")

    ch_c = jnp.where(c == 0, CH0, CH1)
    base = jnp.where(c == 0, s * CH0, NS * CH0 + s * CH1)

    def _zrow(i, _):
        for j in range(8):
            gb[0][i, pl.ds(j * 16, 16)] = jnp.zeros((16,), _f32)
        return 0
    lax.fori_loop(0, 128, _zrow, 0)
    row0 = s * RPT
    for t in range(RPT // 128):
        pltpu.sync_copy(gb[0], zsh.at[pl.ds(row0 + t * 128, 128)])
    plsc.subcore_barrier()

    def _super(t, _):
        off0 = (base + t * 8) * 128
        hidx = [
            (pltpu.async_copy(src_h.at[pl.ds(off0 + b * 128, 128)], six[b], isem[b]),
             pltpu.async_copy(dst_h.at[pl.ds(off0 + b * 128, 128)], dix[b], isem[b]))
            for b in range(8)
        ]
        hg = {}
        hs = {}
        for b in range(8):
            g = b % 2
            if b >= 2:
                hs.pop(b - 2).wait()
            for h in hidx[b]:
                h.wait()
            hg[b] = pltpu.async_copy(y_h.at[six[b]], gb[g], gsem[g])
            if b >= 1:
                hg.pop(b - 1).wait()
                hs[b - 1] = pltpu.async_copy(
                    gb[(b - 1) % 2], zsh.at[dix[b - 1]], ssem[(b - 1) % 2],
                    add=True)
        hg.pop(7).wait()
        hs[7] = pltpu.async_copy(gb[1], zsh.at[dix[7]], ssem[1], add=True)
        hs.pop(6).wait()
        hs.pop(7).wait()
        return 0
    lax.fori_loop(0, ch_c // 8, _super, 0)

    plsc.subcore_barrier()
    pltpu.sync_copy(zsh.at[pl.ds(row0, RPT)],
                    z_h.at[pl.ds(c * NPAD + row0, RPT)])


_spmm = functools.partial(
    pl.kernel,
    out_type=jax.ShapeDtypeStruct((2 * NPAD, D), _f32),
    mesh=_mesh,
    scratch_types=(
        [pltpu.VMEM_SHARED((NPAD, D), _f32)]
        + [pltpu.VMEM((128,), _i32) for _ in range(16)]
        + [pltpu.VMEM((128, D), _f32) for _ in range(2)]
        + [pltpu.SemaphoreType.DMA for _ in range(12)]
    ),
)(_spmm_body)


def _t1_body(dp0, dp1, x_ref, y0_ref, dis_ref, inv_ref):
    i = pl.program_id(0)
    deg = dp0[:, 0:1] + dp1[:, 0:1] + 1.0
    rows = i * BR + lax.broadcasted_iota(_i32, (BR, 1), 0)
    mask = rows < N
    dis = jnp.where(mask, lax.rsqrt(deg), 0.0)
    inv = jnp.where(mask, 1.0 / deg, 0.0)
    y0_ref[...] = x_ref[...] * dis
    dis_ref[...] = dis
    inv_ref[...] = inv


def _t1(degp, x_p):
    return pl.pallas_call(
        _t1_body,
        grid=(NBLK,),
        in_specs=[
            pl.BlockSpec((BR, 128), lambda i: (i, 0)),
            pl.BlockSpec((BR, 128), lambda i: (i + NBLK, 0)),
            pl.BlockSpec((BR, D), lambda i: (i, 0)),
        ],
        out_specs=[
            pl.BlockSpec((BR, D), lambda i: (i, 0)),
            pl.BlockSpec((BR, 1), lambda i: (i, 0)),
            pl.BlockSpec((BR, 1), lambda i: (i, 0)),
        ],
        out_shape=[
            jax.ShapeDtypeStruct((NPAD, D), _f32),
            jax.ShapeDtypeStruct((NPAD, 1), _f32),
            jax.ShapeDtypeStruct((NPAD, 1), _f32),
        ],
    )(degp, degp, x_p)


def _t2_body(z0, z1, y, inv, out):
    out[...] = (z0[...] + z1[...] + y[...]) * inv[...]


def _t2(z, y, inv):
    return pl.pallas_call(
        _t2_body,
        grid=(NBLK,),
        in_specs=[
            pl.BlockSpec((BR, D), lambda i: (i, 0)),
            pl.BlockSpec((BR, D), lambda i: (i + NBLK, 0)),
            pl.BlockSpec((BR, D), lambda i: (i, 0)),
            pl.BlockSpec((BR, 1), lambda i: (i, 0)),
        ],
        out_specs=pl.BlockSpec((BR, D), lambda i: (i, 0)),
        out_shape=jax.ShapeDtypeStruct((NPAD, D), _f32),
    )(z, z, y, inv)


def _t3_body(z0, z1, y, dis, w, b, out):
    x2 = dis[...] * (z0[...] + z1[...] + y[...])
    h = lax.dot_general(x2, w[...], (((1,), (1,)), ((), ())),
                        preferred_element_type=_f32)
    h = jnp.maximum(h + b[...], 0.0)
    out[...] = dis[...] * h


def _t3(z, y, dis, w, b):
    return pl.pallas_call(
        _t3_body,
        grid=(NBLK,),
        in_specs=[
            pl.BlockSpec((BR, D), lambda i: (i, 0)),
            pl.BlockSpec((BR, D), lambda i: (i + NBLK, 0)),
            pl.BlockSpec((BR, D), lambda i: (i, 0)),
            pl.BlockSpec((BR, 1), lambda i: (i, 0)),
            pl.BlockSpec((D, D), lambda i: (0, 0)),
            pl.BlockSpec((1, D), lambda i: (0, 0)),
        ],
        out_specs=pl.BlockSpec((BR, D), lambda i: (i, 0)),
        out_shape=jax.ShapeDtypeStruct((NPAD, D), _f32),
    )(z, z, y, dis, w, b)


def _t5_body(z0, z1, y, dis, w, b, bids, wfc, bfc, out, acc):
    i = pl.program_id(0)

    @pl.when(i == 0)
    def _():
        acc[...] = jnp.zeros_like(acc)

    x2 = dis[...] * (z0[...] + z1[...] + y[...])
    h = lax.dot_general(x2, w[...], (((1,), (1,)), ((), ())),
                        preferred_element_type=_f32)
    h = jnp.maximum(h + b[...], 0.0)
    ids = bids[0, 0, :]
    oh = (ids[None, :] == lax.broadcasted_iota(_i32, (G, BR), 0)).astype(_f32)
    acc[...] += lax.dot_general(oh, h, (((1,), (0,)), ((), ())),
                                preferred_element_type=_f32)

    @pl.when(i == pl.num_programs(0) - 1)
    def _():
        out[...] = lax.dot_general(acc[...], wfc[...],
                                   (((1,), (1,)), ((), ())),
                                   preferred_element_type=_f32) + bfc[...]


def _t5(z, y, dis, w, b, bids, wfc, bfc):
    return pl.pallas_call(
        _t5_body,
        grid=(NBLK,),
        in_specs=[
            pl.BlockSpec((BR, D), lambda i: (i, 0)),
            pl.BlockSpec((BR, D), lambda i: (i + NBLK, 0)),
            pl.BlockSpec((BR, D), lambda i: (i, 0)),
            pl.BlockSpec((BR, 1), lambda i: (i, 0)),
            pl.BlockSpec((D, D), lambda i: (0, 0)),
            pl.BlockSpec((1, D), lambda i: (0, 0)),
            pl.BlockSpec((1, 1, BR), lambda i: (i, 0, 0)),
            pl.BlockSpec((3, D), lambda i: (0, 0)),
            pl.BlockSpec((1, 3), lambda i: (0, 0)),
        ],
        out_specs=pl.BlockSpec((G, 3), lambda i: (0, 0)),
        out_shape=jax.ShapeDtypeStruct((G, 3), _f32),
        scratch_shapes=[pltpu.VMEM((G, D), _f32)],
    )(z, z, y, dis, w, b, bids, wfc, bfc)


def kernel(x, edge_index, batch, W1, b1, W2, b2, Wfc, bfc):
    src = edge_index[0]
    dst = edge_index[1]
    src_p = jnp.concatenate([src, jnp.full((EPAD - E,), N, _i32)])
    dst_p = jnp.concatenate([dst, jnp.full((EPAD - E,), N + 1, _i32)])
    x_p = jnp.pad(x, ((0, NPAD - N), (0, 0)))
    batch_p = jnp.concatenate(
        [batch, jnp.full((NPAD - N,), G, _i32)]).reshape(NBLK, 1, BR)
    b1r = b1.reshape(1, D)
    b2r = b2.reshape(1, D)
    bfcr = bfc.reshape(1, 3)

    src2, dst2, degp = _prep(src_p, dst_p)
    y0, dis, inv = _t1(degp, x_p)

    z = _spmm(src2, dst2, y0)
    y1 = _t2(z, y0, inv)
    z = _spmm(src2, dst2, y1)
    y = _t3(z, y1, dis, W1, b1r)

    z = _spmm(src2, dst2, y)
    y1 = _t2(z, y, inv)
    z = _spmm(src2, dst2, y1)
    return _t5(z, y1, dis, W2, b2r, batch_p, Wfc, bfcr)
